# baseline (speedup 1.0000x reference)
import functools

import jax
import jax.numpy as jnp
from jax import lax
from jax.experimental import pallas as pl
from jax.experimental.pallas import tpu as pltpu
from jax.experimental.pallas import tpu_sc as plsc

TOPK = 5
POOL = 30
PLEN = 8
DIM = 768
B = 1024
HALF = (PLEN // 2) * DIM

NC = 2
NS = 16
NW = NC * NS
ROWS = B * TOPK
RPW = ROWS // NW
CHUNK = 8
NIT = RPW // CHUNK


def _topk_body(xq_ref, ek_ref, idx_ref):
    ek = ek_ref[...]
    norm = jnp.sqrt(jnp.sum(ek * ek, axis=1, keepdims=True))
    ekn = ek / jnp.maximum(norm, 1e-12)
    q = xq_ref[...]
    qn = jnp.sqrt(jnp.sum(q * q, axis=1, keepdims=True))
    q = q / jnp.maximum(qn, 1e-12)
    s = lax.dot_general(
        q, ekn, (((1,), (1,)), ((), ())),
        preferred_element_type=jnp.float32)
    iota = lax.broadcasted_iota(jnp.int32, s.shape, 1)
    cols = []
    for _ in range(TOPK):
        m = jnp.max(s, axis=1, keepdims=True)
        it = jnp.min(jnp.where(s == m, iota, POOL), axis=1, keepdims=True)
        cols.append(it)
        s = jnp.where(iota == it, -jnp.inf, s)
    cols += [jnp.zeros((B, 1), jnp.int32)] * (8 - TOPK)
    idx_ref[...] = jnp.concatenate(cols, axis=1)


_topk = pl.pallas_call(
    _topk_body,
    out_shape=jax.ShapeDtypeStruct((B, 8), jnp.int32),
)


def _gather_body(ekt, evt, idx_hbm, ek_out, ev_out,
                 idx_v, ekb, evb, gsem0, gsem1, wsem0, wsem1):
    wid = lax.axis_index("s") * NC + lax.axis_index("c")
    base = wid * RPW
    pltpu.sync_copy(idx_hbm.at[wid], idx_v)
    gsems = (gsem0, gsem1)
    wsems = (wsem0, wsem1)
    pending = [None, None]
    for j in range(NIT):
        slot = j & 1
        if pending[slot] is not None:
            for d in pending[slot]:
                d.wait()
        g0 = pltpu.async_copy(ekt.at[idx_v.at[j]], ekb.at[slot], gsems[slot])
        g1 = pltpu.async_copy(evt.at[idx_v.at[j]], evb.at[slot], gsems[slot])
        g0.wait()
        g1.wait()
        dst = pl.ds(base + j * CHUNK, CHUNK)
        pending[slot] = (
            pltpu.async_copy(ekb.at[slot], ek_out.at[dst], wsems[slot]),
            pltpu.async_copy(evb.at[slot], ev_out.at[dst], wsems[slot]),
        )
    for slot in (0, 1):
        if pending[slot] is not None:
            for d in pending[slot]:
                d.wait()


@functools.lru_cache(maxsize=1)
def _gather_call():
    return pl.kernel(
        _gather_body,
        out_type=[jax.ShapeDtypeStruct((ROWS, HALF), jnp.float32),
                  jax.ShapeDtypeStruct((ROWS, HALF), jnp.float32)],
        mesh=plsc.VectorSubcoreMesh(
            core_axis_name="c", subcore_axis_name="s",
            num_cores=NC, num_subcores=NS),
        scratch_types=[
            pltpu.VMEM((NIT, CHUNK), jnp.int32),
            pltpu.VMEM((2, CHUNK, HALF), jnp.float32),
            pltpu.VMEM((2, CHUNK, HALF), jnp.float32),
            pltpu.SemaphoreType.DMA,
            pltpu.SemaphoreType.DMA,
            pltpu.SemaphoreType.DMA,
            pltpu.SemaphoreType.DMA,
        ],
    )


def kernel(x_query, x, e_k, e_p, layer_id):
    idx8 = _topk(x_query, e_k)
    idx = idx8[:, :TOPK].reshape(NW, NIT, CHUNK)
    ekt = e_p[:, :PLEN // 2, :].reshape(POOL, HALF)
    evt = e_p[:, PLEN // 2:, :].reshape(POOL, HALF)
    ek_o, ev_o = _gather_call()(ekt, evt, idx)
    shape = (B, TOPK * (PLEN // 2), DIM)
    return (ek_o.reshape(shape), ev_o.reshape(shape),
            jnp.float32(0.0), x)


# speedup vs baseline: 2.4533x; 2.4533x over previous
import functools

import jax
import jax.numpy as jnp
from jax import lax
from jax.experimental import pallas as pl
from jax.experimental.pallas import tpu as pltpu
from jax.experimental.pallas import tpu_sc as plsc

TOPK = 5
POOL = 30
PLEN = 8
DIM = 768
B = 1024
IHALF = PLEN // 2
HALF = IHALF * DIM

NC = 2
NS = 16
NW = NC * NS
BPW = B // NW
CHUNK = 16
NCH = BPW // CHUNK


def _topk_body(xq_ref, ek_ref, idx_ref):
    ek = ek_ref[...]
    norm = jnp.sqrt(jnp.sum(ek * ek, axis=1, keepdims=True))
    ekn = ek / jnp.maximum(norm, 1e-12)
    q = xq_ref[...]
    qn = jnp.sqrt(jnp.sum(q * q, axis=1, keepdims=True))
    q = q / jnp.maximum(qn, 1e-12)
    s = lax.dot_general(
        q, ekn, (((1,), (1,)), ((), ())),
        preferred_element_type=jnp.float32)
    iota = lax.broadcasted_iota(jnp.int32, s.shape, 1)
    cols = []
    for _ in range(TOPK):
        m = jnp.max(s, axis=1, keepdims=True)
        it = jnp.min(jnp.where(s == m, iota, POOL), axis=1, keepdims=True)
        cols.append(it)
        s = jnp.where(iota == it, -jnp.inf, s)
    cols += [jnp.zeros((B, 1), jnp.int32)] * (8 - TOPK)
    idx_ref[...] = jnp.concatenate(cols, axis=1)


_topk = pl.pallas_call(
    _topk_body,
    out_shape=jax.ShapeDtypeStruct((B, 8), jnp.int32),
)


def _gather_body(ekt, evt, idx_hbm, pk_out, pv_out,
                 idx_v, ekb, evb, gsem, wsem_ek, wsem_ev):
    wid = lax.axis_index("s") * NC + lax.axis_index("c")
    b0 = wid * BPW
    pltpu.sync_copy(idx_hbm.at[:, pl.ds(NCH * wid, NCH), :], idx_v)

    def unit(table, buf, wsem, t, c, out):
        g = pltpu.async_copy(table.at[idx_v.at[t, c]], buf, gsem)
        g.wait()
        dst_b = pl.ds(b0 + c * CHUNK, CHUNK)
        return [
            pltpu.async_copy(
                buf.at[:, pl.ds(i * DIM, DIM)],
                out.at[IHALF * t + i, dst_b, :],
                wsem)
            for i in range(IHALF)
        ]

    ek_pend = None
    ev_pend = None
    for t in range(TOPK):
        for c in range(NCH):
            if ek_pend is not None:
                for d in ek_pend:
                    d.wait()
            ek_pend = unit(ekt, ekb, wsem_ek, t, c, pk_out)
            if ev_pend is not None:
                for d in ev_pend:
                    d.wait()
            ev_pend = unit(evt, evb, wsem_ev, t, c, pv_out)
    for d in ek_pend:
        d.wait()
    for d in ev_pend:
        d.wait()


@functools.lru_cache(maxsize=1)
def _gather_call():
    return pl.kernel(
        _gather_body,
        out_type=[jax.ShapeDtypeStruct((TOPK * IHALF, B, DIM), jnp.float32),
                  jax.ShapeDtypeStruct((TOPK * IHALF, B, DIM), jnp.float32)],
        mesh=plsc.VectorSubcoreMesh(
            core_axis_name="c", subcore_axis_name="s",
            num_cores=NC, num_subcores=NS),
        scratch_types=[
            pltpu.VMEM((TOPK, NCH, CHUNK), jnp.int32),
            pltpu.VMEM((CHUNK, HALF), jnp.float32),
            pltpu.VMEM((CHUNK, HALF), jnp.float32),
            pltpu.SemaphoreType.DMA,
            pltpu.SemaphoreType.DMA,
            pltpu.SemaphoreType.DMA,
        ],
    )


def kernel(x_query, x, e_k, e_p, layer_id):
    idx8 = _topk(x_query, e_k)
    idx = idx8[:, :TOPK].T.reshape(TOPK, B // CHUNK, CHUNK)
    ekt = e_p[:, :IHALF, :].reshape(POOL, HALF)
    evt = e_p[:, IHALF:, :].reshape(POOL, HALF)
    pk, pv = _gather_call()(ekt, evt, idx)
    ek_o = jnp.transpose(pk, (1, 0, 2))
    ev_o = jnp.transpose(pv, (1, 0, 2))
    return (ek_o, ev_o, jnp.float32(0.0), x)


# speedup vs baseline: 2.8591x; 1.1654x over previous
import functools

import jax
import jax.numpy as jnp
from jax import lax
from jax.experimental import pallas as pl
from jax.experimental.pallas import tpu as pltpu
from jax.experimental.pallas import tpu_sc as plsc

TOPK = 5
POOL = 30
PLEN = 8
DIM = 768
B = 1024
IHALF = PLEN // 2
HALF = IHALF * DIM

NC = 2
NS = 16
NW = NC * NS
BPW = B // NW
CHUNK = 16
NCH = BPW // CHUNK


def _topk_body(xq_ref, ek_ref, idx_ref):
    ek = ek_ref[...]
    norm = jnp.sqrt(jnp.sum(ek * ek, axis=1, keepdims=True))
    ekn = ek / jnp.maximum(norm, 1e-12)
    q = xq_ref[...]
    qn = jnp.sqrt(jnp.sum(q * q, axis=1, keepdims=True))
    q = q / jnp.maximum(qn, 1e-12)
    s = lax.dot_general(
        q, ekn, (((1,), (1,)), ((), ())),
        preferred_element_type=jnp.float32)
    iota = lax.broadcasted_iota(jnp.int32, s.shape, 1)
    cols = []
    for _ in range(TOPK):
        m = jnp.max(s, axis=1, keepdims=True)
        it = jnp.min(jnp.where(s == m, iota, POOL), axis=1, keepdims=True)
        cols.append(it)
        s = jnp.where(iota == it, -jnp.inf, s)
    cols += [jnp.zeros((B, 1), jnp.int32)] * (8 - TOPK)
    idx_ref[...] = jnp.concatenate(cols, axis=1)


_topk = pl.pallas_call(
    _topk_body,
    out_shape=jax.ShapeDtypeStruct((B, 8), jnp.int32),
)


def _gather_body(ekt, idx_hbm, pk_out,
                 idx_v, buf0, buf1, gsem, wsem0, wsem1):
    wid = lax.axis_index("s") * NC + lax.axis_index("c")
    b0 = wid * BPW
    pltpu.sync_copy(idx_hbm.at[:, pl.ds(NCH * wid, NCH), :], idx_v)

    bufs = (buf0, buf1)
    wsems = (wsem0, wsem1)

    def unit(slot, t, c):
        g = pltpu.async_copy(ekt.at[idx_v.at[t, c]], bufs[slot], gsem)
        g.wait()
        dst_b = pl.ds(b0 + c * CHUNK, CHUNK)
        return [
            pltpu.async_copy(
                bufs[slot].at[:, pl.ds(i * DIM, DIM)],
                pk_out.at[IHALF * t + i, dst_b, :],
                wsems[slot])
            for i in range(IHALF)
        ]

    pend = [None, None]
    for u in range(TOPK * NCH):
        t, c = divmod(u, NCH)
        slot = u & 1
        if pend[slot] is not None:
            for d in pend[slot]:
                d.wait()
        pend[slot] = unit(slot, t, c)
    for slot in (0, 1):
        if pend[slot] is not None:
            for d in pend[slot]:
                d.wait()


@functools.lru_cache(maxsize=1)
def _gather_call():
    return pl.kernel(
        _gather_body,
        out_type=jax.ShapeDtypeStruct((TOPK * IHALF, B, DIM), jnp.float32),
        mesh=plsc.VectorSubcoreMesh(
            core_axis_name="c", subcore_axis_name="s",
            num_cores=NC, num_subcores=NS),
        scratch_types=[
            pltpu.VMEM((TOPK, NCH, CHUNK), jnp.int32),
            pltpu.VMEM((CHUNK, HALF), jnp.float32),
            pltpu.VMEM((CHUNK, HALF), jnp.float32),
            pltpu.SemaphoreType.DMA,
            pltpu.SemaphoreType.DMA,
            pltpu.SemaphoreType.DMA,
        ],
    )


NBLK = 4
BS = B // NBLK


def _evwr_body(idx_ref, tbl_ref, out_ref):
    pool_iota = lax.broadcasted_iota(jnp.int32, (BS, POOL), 1)
    for t in range(TOPK):
        oh = (idx_ref[:, t:t + 1] == pool_iota).astype(jnp.float32)
        for i in range(IHALF):
            seg = lax.dot_general(
                oh, tbl_ref[:, i, :], (((1,), (0,)), ((), ())),
                preferred_element_type=jnp.float32,
                precision=lax.Precision.HIGHEST)
            out_ref[IHALF * t + i, :, :] = seg


_evwriter = pl.pallas_call(
    _evwr_body,
    grid=(NBLK,),
    in_specs=[
        pl.BlockSpec((BS, 8), lambda j: (j, 0)),
        pl.BlockSpec((POOL, IHALF, DIM), lambda j: (0, 0, 0)),
    ],
    out_specs=pl.BlockSpec((TOPK * IHALF, BS, DIM), lambda j: (0, j, 0)),
    out_shape=jax.ShapeDtypeStruct((TOPK * IHALF, B, DIM), jnp.float32),
)


def kernel(x_query, x, e_k, e_p, layer_id):
    idx8 = _topk(x_query, e_k)
    idx = idx8[:, :TOPK].T.reshape(TOPK, B // CHUNK, CHUNK)
    ekt = e_p[:, :IHALF, :].reshape(POOL, HALF)
    pk = _gather_call()(ekt, idx)
    pv = _evwriter(idx8, e_p[:, IHALF:, :])
    ek_o = jnp.transpose(pk, (1, 0, 2))
    ev_o = jnp.transpose(pv, (1, 0, 2))
    return (ek_o, ev_o, jnp.float32(0.0), x)


# speedup vs baseline: 2.8802x; 1.0074x over previous
import functools

import jax
import jax.numpy as jnp
from jax import lax
from jax.experimental import pallas as pl
from jax.experimental.pallas import tpu as pltpu
from jax.experimental.pallas import tpu_sc as plsc

TOPK = 5
POOL = 30
PLEN = 8
DIM = 768
B = 1024
IHALF = PLEN // 2
HALF = IHALF * DIM

NC = 2
NS = 16
NW = NC * NS
BPW = B // NW
CHUNK = 16
NCH = BPW // CHUNK


def _topk_body(xq_ref, ek_ref, idx_ref):
    ek = ek_ref[...]
    norm = jnp.sqrt(jnp.sum(ek * ek, axis=1, keepdims=True))
    ekn = ek / jnp.maximum(norm, 1e-12)
    q = xq_ref[...]
    qn = jnp.sqrt(jnp.sum(q * q, axis=1, keepdims=True))
    q = q / jnp.maximum(qn, 1e-12)
    s = lax.dot_general(
        q, ekn, (((1,), (1,)), ((), ())),
        preferred_element_type=jnp.float32)
    iota = lax.broadcasted_iota(jnp.int32, s.shape, 1)
    cols = []
    for _ in range(TOPK):
        m = jnp.max(s, axis=1, keepdims=True)
        it = jnp.min(jnp.where(s == m, iota, POOL), axis=1, keepdims=True)
        cols.append(it)
        s = jnp.where(iota == it, -jnp.inf, s)
    cols += [jnp.zeros((B, 1), jnp.int32)] * (8 - TOPK)
    idx_ref[...] = jnp.concatenate(cols, axis=1)


_topk = pl.pallas_call(
    _topk_body,
    out_shape=jax.ShapeDtypeStruct((B, 8), jnp.int32),
)


def _gather_body(ekt, idx_hbm, pk_out,
                 idx_v, buf0, buf1, gsem0, gsem1, wsem0, wsem1):
    wid = lax.axis_index("s") * NC + lax.axis_index("c")
    b0 = wid * BPW
    pltpu.sync_copy(idx_hbm.at[:, pl.ds(NCH * wid, NCH), :], idx_v)

    bufs = (buf0, buf1)
    gsems = (gsem0, gsem1)
    wsems = (wsem0, wsem1)
    NU = TOPK * NCH

    def start_gather(u, slot):
        t, c = divmod(u, NCH)
        return pltpu.async_copy(ekt.at[idx_v.at[t, c]], bufs[slot], gsems[slot])

    def start_writes(u, slot):
        t, c = divmod(u, NCH)
        dst_b = pl.ds(b0 + c * CHUNK, CHUNK)
        return [
            pltpu.async_copy(
                bufs[slot].at[:, pl.ds(i * DIM, DIM)],
                pk_out.at[IHALF * t + i, dst_b, :],
                wsems[slot])
            for i in range(IHALF)
        ]

    pend_w = [None, None]
    g = [None, None]
    g[0] = start_gather(0, 0)
    for u in range(NU):
        slot = u & 1
        nxt = slot ^ 1
        if u + 1 < NU:
            if pend_w[nxt] is not None:
                for d in pend_w[nxt]:
                    d.wait()
                pend_w[nxt] = None
            g[nxt] = start_gather(u + 1, nxt)
        g[slot].wait()
        pend_w[slot] = start_writes(u, slot)
    for slot in (0, 1):
        if pend_w[slot] is not None:
            for d in pend_w[slot]:
                d.wait()


@functools.lru_cache(maxsize=1)
def _gather_call():
    return pl.kernel(
        _gather_body,
        out_type=jax.ShapeDtypeStruct((TOPK * IHALF, B, DIM), jnp.float32),
        mesh=plsc.VectorSubcoreMesh(
            core_axis_name="c", subcore_axis_name="s",
            num_cores=NC, num_subcores=NS),
        scratch_types=[
            pltpu.VMEM((TOPK, NCH, CHUNK), jnp.int32),
            pltpu.VMEM((CHUNK, HALF), jnp.float32),
            pltpu.VMEM((CHUNK, HALF), jnp.float32),
            pltpu.SemaphoreType.DMA,
            pltpu.SemaphoreType.DMA,
            pltpu.SemaphoreType.DMA,
            pltpu.SemaphoreType.DMA,
        ],
    )


NBLK = 4
BS = B // NBLK


def _evwr_body(idx_ref, tbl_ref, out_ref):
    pool_iota = lax.broadcasted_iota(jnp.int32, (BS, POOL), 1)
    for t in range(TOPK):
        oh = (idx_ref[:, t:t + 1] == pool_iota).astype(jnp.float32)
        for i in range(IHALF):
            seg = lax.dot_general(
                oh, tbl_ref[:, i, :], (((1,), (0,)), ((), ())),
                preferred_element_type=jnp.float32,
                precision=lax.Precision.HIGHEST)
            out_ref[IHALF * t + i, :, :] = seg


_evwriter = pl.pallas_call(
    _evwr_body,
    grid=(NBLK,),
    in_specs=[
        pl.BlockSpec((BS, 8), lambda j: (j, 0)),
        pl.BlockSpec((POOL, IHALF, DIM), lambda j: (0, 0, 0)),
    ],
    out_specs=pl.BlockSpec((TOPK * IHALF, BS, DIM), lambda j: (0, j, 0)),
    out_shape=jax.ShapeDtypeStruct((TOPK * IHALF, B, DIM), jnp.float32),
)


def kernel(x_query, x, e_k, e_p, layer_id):
    idx8 = _topk(x_query, e_k)
    idx = idx8[:, :TOPK].T.reshape(TOPK, B // CHUNK, CHUNK)
    ekt = e_p[:, :IHALF, :].reshape(POOL, HALF)
    pk = _gather_call()(ekt, idx)
    pv = _evwriter(idx8, e_p[:, IHALF:, :])
    ek_o = jnp.transpose(pk, (1, 0, 2))
    ev_o = jnp.transpose(pv, (1, 0, 2))
    return (ek_o, ev_o, jnp.float32(0.0), x)
